# Initial kernel scaffold; baseline (speedup 1.0000x reference)
import jax
import jax.numpy as jnp
from jax.experimental import pallas as pl


def kernel(step, lprobs, scores):
    raise NotImplementedError("write your pallas kernel here")


# speedup vs baseline: 1.1397x; 1.1397x over previous
import functools

import jax
import jax.numpy as jnp
from jax import lax
from jax.experimental import pallas as pl
from jax.experimental.pallas import tpu as pltpu
from jax.experimental.pallas import tpu_sc as plsc

NC = 2
NS = 16
NW = NC * NS
L = 16

BSZ = 128
VOCAB = 100000
VK = 8
ROWS_PER_W = BSZ // NW
NCHUNK = VOCAB // L
CPB = 50
NBLK = NCHUNK // CPB
BLK = CPB * L
MAX_SLOTS = 512

NEG_INF = float("-inf")
BIG_I32 = 2**31 - 1


def _topk_body(step_hbm, lprobs_hbm, scores_hbm,
               out_val_hbm, out_idx_hbm, out_beam_hbm,
               row_v, cand_val, cand_idx, st_val, st_idx, st_beam,
               step_v, scores_v, bm_smem, slot_smem):
    wid = lax.axis_index("s") * NC + lax.axis_index("c")
    lane = lax.iota(jnp.int32, L)
    lanef = lane.astype(jnp.float32)
    minus_inf = jnp.full((L,), NEG_INF, jnp.float32)
    plus_inf = jnp.full((L,), float("inf"), jnp.float32)
    big_vec = jnp.full((L,), BIG_I32, jnp.int32)
    zero_i = jnp.zeros((L,), jnp.int32)

    pltpu.sync_copy(step_hbm, step_v)
    pltpu.sync_copy(scores_hbm, scores_v)
    ssum = jnp.zeros((L,), jnp.float32)
    for i in range(BSZ * 4 // L):
        ssum = ssum + scores_v[pl.ds(i * L, L)]
    mean = jnp.sum(ssum) * (1.0 / (BSZ * 4))
    stepf = jnp.max(step_v[...].astype(jnp.float32))
    shift = stepf * mean

    for rr in range(ROWS_PER_W):
        r = wid * ROWS_PER_W + rr
        pltpu.sync_copy(lprobs_hbm.at[r, 0], row_v)
        v0 = row_v[pl.ds(0, L)]
        row_v[pl.ds(0, L)] = jnp.where(lane == 0, minus_inf, v0)

        def blk_body(b, top8):
            base = b * BLK
            acc = minus_inf
            for j in range(CPB):
                acc = jnp.maximum(acc, row_v[pl.ds(base + j * L, L)])
            bm_smem[b] = jnp.max(acc)
            accs = plsc.sort_key_val(acc, lane, descending=True)[0]
            comb = jnp.where(lane < VK, top8, lax.rev(accs, (0,)))
            return plsc.sort_key_val(comb, lane, descending=True)[0]

        top8 = lax.fori_loop(0, NBLK, blk_body, minus_inf)
        t = jnp.min(jnp.where(lane < VK, top8, plus_inf))
        t_vec = jnp.full((L,), t, jnp.float32)

        slot_smem[0] = 0

        def p2_body(b, carry):
            @pl.when(bm_smem[b] >= t)
            def _visit():
                base = b * BLK
                for j in range(CPB):
                    v = row_v[pl.ds(base + j * L, L)]
                    cm = jnp.max(v)
                    s = slot_smem[0]

                    @pl.when((cm >= t) & (s < MAX_SLOTS))
                    def _store():
                        cand_val[pl.ds(s * L, L)] = v
                        cand_idx[pl.ds(s * L, L)] = lane + (base + j * L)
                        slot_smem[0] = s + 1
            return carry

        lax.fori_loop(0, NBLK, p2_body, 0)
        nslots = slot_smem[0]

        ovv = minus_inf
        oiv = zero_i
        for k in range(VK):
            def mx_body(s, macc):
                return jnp.maximum(macc, cand_val[pl.ds(s * L, L)])

            m = jnp.max(lax.fori_loop(0, nslots, mx_body, minus_inf))
            m_vec = jnp.full((L,), m, jnp.float32)

            def ix_body(s, iacc):
                cv = cand_val[pl.ds(s * L, L)]
                ci = cand_idx[pl.ds(s * L, L)]
                return jnp.minimum(iacc, jnp.where(cv == m_vec, ci, big_vec))

            ci = jnp.min(lax.fori_loop(0, nslots, ix_body, big_vec))
            ci_vec = jnp.full((L,), ci, jnp.int32)

            def rm_body(s, carry):
                cv = cand_val[pl.ds(s * L, L)]
                cand_val[pl.ds(s * L, L)] = jnp.where(
                    cand_idx[pl.ds(s * L, L)] == ci_vec, minus_inf, cv)
                return carry

            lax.fori_loop(0, nslots, rm_body, 0)
            ovv = jnp.where(lane == k, m_vec, ovv)
            oiv = jnp.where(lane == k, ci_vec, oiv)

        sh_vec = jnp.full((L,), shift, jnp.float32)
        st_val[pl.ds(rr * L, L)] = ovv + sh_vec
        st_idx[pl.ds(rr * L, L)] = oiv
        st_beam[pl.ds(rr * L, L)] = zero_i

    base = wid * ROWS_PER_W * L
    pltpu.sync_copy(st_val, out_val_hbm.at[pl.ds(base, ROWS_PER_W * L)])
    pltpu.sync_copy(st_idx, out_idx_hbm.at[pl.ds(base, ROWS_PER_W * L)])
    pltpu.sync_copy(st_beam, out_beam_hbm.at[pl.ds(base, ROWS_PER_W * L)])


@jax.jit
def _sc_topk(step_v, lprobs, scores_flat):
    mesh = plsc.VectorSubcoreMesh(
        core_axis_name="c", subcore_axis_name="s",
        num_cores=NC, num_subcores=NS)
    fn = pl.kernel(
        _topk_body,
        out_type=(
            jax.ShapeDtypeStruct((BSZ * L,), jnp.float32),
            jax.ShapeDtypeStruct((BSZ * L,), jnp.int32),
            jax.ShapeDtypeStruct((BSZ * L,), jnp.int32),
        ),
        mesh=mesh,
        compiler_params=pltpu.CompilerParams(needs_layout_passes=False),
        scratch_types=[
            pltpu.VMEM((VOCAB,), jnp.float32),
            pltpu.VMEM((MAX_SLOTS * L,), jnp.float32),
            pltpu.VMEM((MAX_SLOTS * L,), jnp.int32),
            pltpu.VMEM((ROWS_PER_W * L,), jnp.float32),
            pltpu.VMEM((ROWS_PER_W * L,), jnp.int32),
            pltpu.VMEM((ROWS_PER_W * L,), jnp.int32),
            pltpu.VMEM((L,), jnp.int32),
            pltpu.VMEM((BSZ * 4,), jnp.float32),
            pltpu.SMEM((NBLK,), jnp.float32),
            pltpu.SMEM((1,), jnp.int32),
        ],
    )
    return fn(step_v, lprobs, scores_flat)


def kernel(step, lprobs, scores):
    step_v = jnp.broadcast_to(
        jnp.asarray(step, jnp.int32).reshape(()), (L,))
    sc, ix, bm = _sc_topk(step_v, lprobs, scores.reshape(-1))
    return (sc.reshape(BSZ, L)[:, :VK], ix.reshape(BSZ, L)[:, :VK],
            bm.reshape(BSZ, L)[:, :VK])


# speedup vs baseline: 1.5323x; 1.3445x over previous
import functools

import jax
import jax.numpy as jnp
from jax import lax
from jax.experimental import pallas as pl
from jax.experimental.pallas import tpu as pltpu
from jax.experimental.pallas import tpu_sc as plsc

NC = 2
NS = 16
NW = NC * NS
L = 16

BSZ = 128
VOCAB = 100000
VK = 8
ROWS_PER_W = BSZ // NW
NCHUNK = VOCAB // L
CPB = 50
NBLK = NCHUNK // CPB
BLK = CPB * L
MAX_SLOTS = 512

NEG_INF = float("-inf")
BIG_I32 = 2**31 - 1


def _topk_body(step_hbm, lprobs_hbm, scores_hbm,
               out_val_hbm, out_idx_hbm, out_beam_hbm,
               row_v, cand_val, cand_idx, st_val, st_idx, st_beam,
               step_v, scores_v, bm_smem, slot_smem):
    wid = lax.axis_index("s") * NC + lax.axis_index("c")
    lane = lax.iota(jnp.int32, L)
    lanef = lane.astype(jnp.float32)
    minus_inf = jnp.full((L,), NEG_INF, jnp.float32)
    plus_inf = jnp.full((L,), float("inf"), jnp.float32)
    big_vec = jnp.full((L,), BIG_I32, jnp.int32)
    zero_i = jnp.zeros((L,), jnp.int32)

    pltpu.sync_copy(step_hbm, step_v)
    pltpu.sync_copy(scores_hbm, scores_v)
    ssum = jnp.zeros((L,), jnp.float32)
    for i in range(BSZ * 4 // L):
        ssum = ssum + scores_v[pl.ds(i * L, L)]
    mean = jnp.sum(ssum) * (1.0 / (BSZ * 4))
    stepf = jnp.max(step_v[...].astype(jnp.float32))
    shift = stepf * mean

    for rr in range(ROWS_PER_W):
        r = wid * ROWS_PER_W + rr
        pltpu.sync_copy(lprobs_hbm.at[pl.ds(r * VOCAB, VOCAB)], row_v)
        v0 = row_v[pl.ds(0, L)]
        row_v[pl.ds(0, L)] = jnp.where(lane == 0, minus_inf, v0)

        def blk_body(b, top8):
            base = b * BLK
            acc = minus_inf
            for j in range(CPB):
                acc = jnp.maximum(acc, row_v[pl.ds(base + j * L, L)])
            bm_smem[b] = jnp.max(acc)
            accs = plsc.sort_key_val(acc, lane, descending=True)[0]
            comb = jnp.where(lane < VK, top8, lax.rev(accs, (0,)))
            return plsc.sort_key_val(comb, lane, descending=True)[0]

        top8 = lax.fori_loop(0, NBLK, blk_body, minus_inf)
        t = jnp.min(jnp.where(lane < VK, top8, plus_inf))
        t_vec = jnp.full((L,), t, jnp.float32)

        slot_smem[0] = 0

        def p2_body(b, carry):
            @pl.when(bm_smem[b] >= t)
            def _visit():
                base = b * BLK
                for j in range(CPB):
                    v = row_v[pl.ds(base + j * L, L)]
                    cm = jnp.max(v)
                    s = slot_smem[0]

                    @pl.when((cm >= t) & (s < MAX_SLOTS))
                    def _store():
                        cand_val[pl.ds(s * L, L)] = v
                        cand_idx[pl.ds(s * L, L)] = lane + (base + j * L)
                        slot_smem[0] = s + 1
            return carry

        lax.fori_loop(0, NBLK, p2_body, 0)
        nslots = slot_smem[0]

        ovv = minus_inf
        oiv = zero_i
        for k in range(VK):
            def mx_body(s, macc):
                return jnp.maximum(macc, cand_val[pl.ds(s * L, L)])

            m = jnp.max(lax.fori_loop(0, nslots, mx_body, minus_inf))
            m_vec = jnp.full((L,), m, jnp.float32)

            def ix_body(s, iacc):
                cv = cand_val[pl.ds(s * L, L)]
                ci = cand_idx[pl.ds(s * L, L)]
                return jnp.minimum(iacc, jnp.where(cv == m_vec, ci, big_vec))

            ci = jnp.min(lax.fori_loop(0, nslots, ix_body, big_vec))
            ci_vec = jnp.full((L,), ci, jnp.int32)

            def rm_body(s, carry):
                cv = cand_val[pl.ds(s * L, L)]
                cand_val[pl.ds(s * L, L)] = jnp.where(
                    cand_idx[pl.ds(s * L, L)] == ci_vec, minus_inf, cv)
                return carry

            lax.fori_loop(0, nslots, rm_body, 0)
            ovv = jnp.where(lane == k, m_vec, ovv)
            oiv = jnp.where(lane == k, ci_vec, oiv)

        sh_vec = jnp.full((L,), shift, jnp.float32)
        st_val[pl.ds(rr * L, L)] = ovv + sh_vec
        st_idx[pl.ds(rr * L, L)] = oiv
        st_beam[pl.ds(rr * L, L)] = zero_i

    base = wid * ROWS_PER_W * L
    pltpu.sync_copy(st_val, out_val_hbm.at[pl.ds(base, ROWS_PER_W * L)])
    pltpu.sync_copy(st_idx, out_idx_hbm.at[pl.ds(base, ROWS_PER_W * L)])
    pltpu.sync_copy(st_beam, out_beam_hbm.at[pl.ds(base, ROWS_PER_W * L)])


@jax.jit
def _sc_topk(step_v, lprobs, scores_flat):
    mesh = plsc.VectorSubcoreMesh(
        core_axis_name="c", subcore_axis_name="s",
        num_cores=NC, num_subcores=NS)
    fn = pl.kernel(
        _topk_body,
        out_type=(
            jax.ShapeDtypeStruct((BSZ * L,), jnp.float32),
            jax.ShapeDtypeStruct((BSZ * L,), jnp.int32),
            jax.ShapeDtypeStruct((BSZ * L,), jnp.int32),
        ),
        mesh=mesh,
        compiler_params=pltpu.CompilerParams(needs_layout_passes=False),
        scratch_types=[
            pltpu.VMEM((VOCAB,), jnp.float32),
            pltpu.VMEM((MAX_SLOTS * L,), jnp.float32),
            pltpu.VMEM((MAX_SLOTS * L,), jnp.int32),
            pltpu.VMEM((ROWS_PER_W * L,), jnp.float32),
            pltpu.VMEM((ROWS_PER_W * L,), jnp.int32),
            pltpu.VMEM((ROWS_PER_W * L,), jnp.int32),
            pltpu.VMEM((L,), jnp.int32),
            pltpu.VMEM((BSZ * 4,), jnp.float32),
            pltpu.SMEM((NBLK,), jnp.float32),
            pltpu.SMEM((1,), jnp.int32),
        ],
    )
    return fn(step_v, lprobs, scores_flat)


def kernel(step, lprobs, scores):
    step_v = jnp.broadcast_to(
        jnp.asarray(step, jnp.int32).reshape(()), (L,))
    lp0 = lprobs[:, 0, :].reshape(BSZ * VOCAB)
    sc, ix, bm = _sc_topk(step_v, lp0, scores.reshape(-1))
    return (sc.reshape(BSZ, L)[:, :VK], ix.reshape(BSZ, L)[:, :VK],
            bm.reshape(BSZ, L)[:, :VK])


# speedup vs baseline: 1.5668x; 1.0225x over previous
import functools

import jax
import jax.numpy as jnp
from jax import lax
from jax.experimental import pallas as pl
from jax.experimental.pallas import tpu as pltpu
from jax.experimental.pallas import tpu_sc as plsc

NC = 2
NS = 16
NW = NC * NS
L = 16

BSZ = 128
VOCAB = 100000
VK = 8
ROWS_PER_W = BSZ // NW
NCHUNK = VOCAB // L
CPB = 50
NBLK = NCHUNK // CPB
BLK = CPB * L
MAX_SLOTS = 512

NEG_INF = float("-inf")
BIG_I32 = 2**31 - 1


def _topk_body(step_hbm, lprobs_hbm, scores_hbm,
               out_val_hbm, out_idx_hbm, out_beam_hbm,
               row_v, cand_val, cand_idx, st_val, st_idx, st_beam,
               step_v, scores_v, bm_smem, slot_smem):
    wid = lax.axis_index("s") * NC + lax.axis_index("c")
    lane = lax.iota(jnp.int32, L)
    lanef = lane.astype(jnp.float32)
    minus_inf = jnp.full((L,), NEG_INF, jnp.float32)
    plus_inf = jnp.full((L,), float("inf"), jnp.float32)
    big_vec = jnp.full((L,), BIG_I32, jnp.int32)
    zero_i = jnp.zeros((L,), jnp.int32)

    pltpu.sync_copy(step_hbm, step_v)
    pltpu.sync_copy(scores_hbm, scores_v)
    ssum = jnp.zeros((L,), jnp.float32)
    for i in range(BSZ * 4 // L):
        ssum = ssum + scores_v[pl.ds(i * L, L)]
    mean = jnp.sum(ssum) * (1.0 / (BSZ * 4))
    stepf = jnp.max(step_v[...].astype(jnp.float32))
    shift = stepf * mean

    for rr in range(ROWS_PER_W):
        r = wid * ROWS_PER_W + rr
        with jax.named_scope("row_dma"):
            pltpu.sync_copy(lprobs_hbm.at[pl.ds(r * VOCAB, VOCAB)], row_v)
        v0 = row_v[pl.ds(0, L)]
        row_v[pl.ds(0, L)] = jnp.where(lane == 0, minus_inf, v0)

        def blk_body(b, top8):
            base = b * BLK
            acc = minus_inf
            for j in range(CPB):
                acc = jnp.maximum(acc, row_v[pl.ds(base + j * L, L)])
            bm_smem[b] = jnp.max(acc)
            accs = plsc.sort_key_val(acc, lane, descending=True)[0]
            comb = jnp.where(lane < VK, top8, lax.rev(accs, (0,)))
            return plsc.sort_key_val(comb, lane, descending=True)[0]

        with jax.named_scope("phase1"):
            top8 = lax.fori_loop(0, NBLK, blk_body, minus_inf)
        t = jnp.min(jnp.where(lane < VK, top8, plus_inf))
        t_vec = jnp.full((L,), t, jnp.float32)

        slot_smem[0] = 0

        def p2_body(b, carry):
            @pl.when(bm_smem[b] >= t)
            def _visit():
                base = b * BLK
                for j in range(CPB):
                    v = row_v[pl.ds(base + j * L, L)]
                    cm = jnp.max(v)
                    s = slot_smem[0]

                    @pl.when((cm >= t) & (s < MAX_SLOTS))
                    def _store():
                        cand_val[pl.ds(s * L, L)] = v
                        cand_idx[pl.ds(s * L, L)] = lane + (base + j * L)
                        slot_smem[0] = s + 1
            return carry

        with jax.named_scope("phase2"):
            lax.fori_loop(0, NBLK, p2_body, 0)
        nslots = slot_smem[0]

        ph3 = jax.named_scope("phase3")
        ph3.__enter__()
        ovv = minus_inf
        oiv = zero_i
        for k in range(VK):
            def mx_body(s, macc):
                return jnp.maximum(macc, cand_val[pl.ds(s * L, L)])

            m = jnp.max(lax.fori_loop(0, nslots, mx_body, minus_inf))
            m_vec = jnp.full((L,), m, jnp.float32)

            def ix_body(s, iacc):
                cv = cand_val[pl.ds(s * L, L)]
                ci = cand_idx[pl.ds(s * L, L)]
                return jnp.minimum(iacc, jnp.where(cv == m_vec, ci, big_vec))

            ci = jnp.min(lax.fori_loop(0, nslots, ix_body, big_vec))
            ci_vec = jnp.full((L,), ci, jnp.int32)

            def rm_body(s, carry):
                cv = cand_val[pl.ds(s * L, L)]
                cand_val[pl.ds(s * L, L)] = jnp.where(
                    cand_idx[pl.ds(s * L, L)] == ci_vec, minus_inf, cv)
                return carry

            lax.fori_loop(0, nslots, rm_body, 0)
            ovv = jnp.where(lane == k, m_vec, ovv)
            oiv = jnp.where(lane == k, ci_vec, oiv)

        sh_vec = jnp.full((L,), shift, jnp.float32)
        st_val[pl.ds(rr * L, L)] = ovv + sh_vec
        ph3.__exit__(None, None, None)
        st_idx[pl.ds(rr * L, L)] = oiv
        st_beam[pl.ds(rr * L, L)] = zero_i

    base = wid * ROWS_PER_W * L
    pltpu.sync_copy(st_val, out_val_hbm.at[pl.ds(base, ROWS_PER_W * L)])
    pltpu.sync_copy(st_idx, out_idx_hbm.at[pl.ds(base, ROWS_PER_W * L)])
    pltpu.sync_copy(st_beam, out_beam_hbm.at[pl.ds(base, ROWS_PER_W * L)])


@jax.jit
def _sc_topk(step_v, lprobs, scores_flat):
    mesh = plsc.VectorSubcoreMesh(
        core_axis_name="c", subcore_axis_name="s",
        num_cores=NC, num_subcores=NS)
    fn = pl.kernel(
        _topk_body,
        out_type=(
            jax.ShapeDtypeStruct((BSZ * L,), jnp.float32),
            jax.ShapeDtypeStruct((BSZ * L,), jnp.int32),
            jax.ShapeDtypeStruct((BSZ * L,), jnp.int32),
        ),
        mesh=mesh,
        compiler_params=pltpu.CompilerParams(needs_layout_passes=False),
        scratch_types=[
            pltpu.VMEM((VOCAB,), jnp.float32),
            pltpu.VMEM((MAX_SLOTS * L,), jnp.float32),
            pltpu.VMEM((MAX_SLOTS * L,), jnp.int32),
            pltpu.VMEM((ROWS_PER_W * L,), jnp.float32),
            pltpu.VMEM((ROWS_PER_W * L,), jnp.int32),
            pltpu.VMEM((ROWS_PER_W * L,), jnp.int32),
            pltpu.VMEM((L,), jnp.int32),
            pltpu.VMEM((BSZ * 4,), jnp.float32),
            pltpu.SMEM((NBLK,), jnp.float32),
            pltpu.SMEM((1,), jnp.int32),
        ],
    )
    return fn(step_v, lprobs, scores_flat)


def kernel(step, lprobs, scores):
    step_v = jnp.broadcast_to(
        jnp.asarray(step, jnp.int32).reshape(()), (L,))
    lp0 = lprobs[:, 0, :].reshape(BSZ * VOCAB)
    sc, ix, bm = _sc_topk(step_v, lp0, scores.reshape(-1))
    return (sc.reshape(BSZ, L)[:, :VK], ix.reshape(BSZ, L)[:, :VK],
            bm.reshape(BSZ, L)[:, :VK])


# speedup vs baseline: 1.7257x; 1.1014x over previous
import jax
import jax.numpy as jnp
from jax import lax
from jax.experimental import pallas as pl
from jax.experimental.pallas import tpu as pltpu
from jax.experimental.pallas import tpu_sc as plsc

NC = 2
NS = 16
NW = NC * NS
L = 16

BSZ = 128
VOCAB = 100000
VK = 8
ROWS_PER_W = BSZ // NW
NCHUNK = VOCAB // L
CPB = 50
NBLK = NCHUNK // CPB
BLK = CPB * L
MAX_SLOTS = 1024

NEG_INF = float("-inf")
BIG_I32 = 2**31 - 1
GROUPS = [(0, 16), (16, 16), (32, 16), (48, 2)]


def _topk_body(step_hbm, lprobs_hbm, scores_hbm,
               out_val_hbm, out_idx_hbm, out_beam_hbm,
               row_v, cidx_v, st_val, st_idx, st_beam,
               step_v, scores_v, bm_smem, slot_smem):
    wid = lax.axis_index("s") * NC + lax.axis_index("c")
    lane = lax.iota(jnp.int32, L)
    minus_inf = jnp.full((L,), NEG_INF, jnp.float32)
    plus_inf = jnp.full((L,), float("inf"), jnp.float32)
    big_vec = jnp.full((L,), BIG_I32, jnp.int32)
    zero_i = jnp.zeros((L,), jnp.int32)

    pltpu.sync_copy(step_hbm, step_v)
    pltpu.sync_copy(scores_hbm, scores_v)
    ssum = jnp.zeros((L,), jnp.float32)
    for i in range(BSZ * 4 // L):
        ssum = ssum + scores_v[pl.ds(i * L, L)]
    mean = jnp.sum(ssum) * (1.0 / (BSZ * 4))
    stepf = jnp.max(step_v[...].astype(jnp.float32))
    shift = stepf * mean

    def row_body(rr, row_carry):
        r = wid * ROWS_PER_W + rr
        with jax.named_scope("row_dma"):
            pltpu.sync_copy(lprobs_hbm.at[pl.ds(r * VOCAB, VOCAB)], row_v)
        v0 = row_v[pl.ds(0, L)]
        row_v[pl.ds(0, L)] = jnp.where(lane == 0, minus_inf, v0)

        with jax.named_scope("phase1"):
            def blk_body(b, rowacc):
                base = b * BLK
                acc = minus_inf
                for j in range(CPB):
                    acc = jnp.maximum(acc, row_v[pl.ds(base + j * L, L)])
                bm_smem[b] = jnp.max(acc)
                return jnp.maximum(rowacc, acc)

            rowacc = lax.fori_loop(0, NBLK, blk_body, minus_inf)

        srt = plsc.sort_key_val(rowacc, lane, descending=True)[0]
        t = jnp.min(jnp.where(lane < VK, srt, plus_inf))
        t_vec = jnp.full((L,), t, jnp.float32)

        slot_smem[0] = 0

        with jax.named_scope("phase2"):
            def p2_body(b, carry):
                @pl.when(bm_smem[b] >= t)
                def _visit():
                    cbase = b * CPB
                    for g0, gsz in GROUPS:
                        hits = zero_i
                        for j in range(gsz):
                            v = row_v[pl.ds((cbase + g0 + j) * L, L)]
                            cnt = plsc.all_reduce_population_count(
                                v >= t_vec)
                            hits = jnp.where(lane == j, cnt, hits)
                        hmask = hits > 0
                        s = slot_smem[0]

                        @pl.when(s < MAX_SLOTS - L)
                        def _st():
                            plsc.store_compressed(
                                cidx_v.at[pl.ds(s, L)],
                                lane + (cbase + g0), mask=hmask)
                            slot_smem[0] = s + jnp.max(
                                plsc.all_reduce_population_count(hmask))
                return carry

            lax.fori_loop(0, NBLK, p2_body, 0)
        nslots = slot_smem[0]
        cidx_v[pl.ds(nslots, L)] = zero_i
        ngroups = (nslots + L - 1) // L

        ph3 = jax.named_scope("phase3")
        ph3.__enter__()
        ovv = minus_inf
        oiv = zero_i
        for k in range(VK):
            def mx_body(g, macc):
                base16 = cidx_v[pl.ds(g * L, L)] * L
                for j in range(L):
                    macc = jnp.maximum(
                        macc, plsc.load_gather(row_v, [base16 + j]))
                return macc

            m = jnp.max(lax.fori_loop(0, ngroups, mx_body, minus_inf))
            m_vec = jnp.full((L,), m, jnp.float32)

            def ix_body(g, iacc):
                base16 = cidx_v[pl.ds(g * L, L)] * L
                for j in range(L):
                    xj = plsc.load_gather(row_v, [base16 + j])
                    iacc = jnp.minimum(
                        iacc, jnp.where(xj == m_vec, base16 + j, big_vec))
                return iacc

            ci = jnp.min(lax.fori_loop(0, ngroups, ix_body, big_vec))
            ci_vec = jnp.full((L,), ci, jnp.int32)

            cch = ci // L
            vb = row_v[pl.ds(cch * L, L)]
            row_v[pl.ds(cch * L, L)] = jnp.where(
                cch * L + lane == ci_vec, minus_inf, vb)

            ovv = jnp.where(lane == k, m_vec, ovv)
            oiv = jnp.where(lane == k, ci_vec, oiv)

        sh_vec = jnp.full((L,), shift, jnp.float32)
        st_val[pl.ds(rr * L, L)] = ovv + sh_vec
        st_idx[pl.ds(rr * L, L)] = oiv
        st_beam[pl.ds(rr * L, L)] = zero_i
        ph3.__exit__(None, None, None)
        return row_carry

    lax.fori_loop(0, ROWS_PER_W, row_body, 0)

    base = wid * ROWS_PER_W * L
    pltpu.sync_copy(st_val, out_val_hbm.at[pl.ds(base, ROWS_PER_W * L)])
    pltpu.sync_copy(st_idx, out_idx_hbm.at[pl.ds(base, ROWS_PER_W * L)])
    pltpu.sync_copy(st_beam, out_beam_hbm.at[pl.ds(base, ROWS_PER_W * L)])


@jax.jit
def _sc_topk(step_v, lprobs, scores_flat):
    mesh = plsc.VectorSubcoreMesh(
        core_axis_name="c", subcore_axis_name="s",
        num_cores=NC, num_subcores=NS)
    fn = pl.kernel(
        _topk_body,
        out_type=(
            jax.ShapeDtypeStruct((BSZ * L,), jnp.float32),
            jax.ShapeDtypeStruct((BSZ * L,), jnp.int32),
            jax.ShapeDtypeStruct((BSZ * L,), jnp.int32),
        ),
        mesh=mesh,
        compiler_params=pltpu.CompilerParams(needs_layout_passes=False),
        scratch_types=[
            pltpu.VMEM((VOCAB,), jnp.float32),
            pltpu.VMEM((MAX_SLOTS + L,), jnp.int32),
            pltpu.VMEM((ROWS_PER_W * L,), jnp.float32),
            pltpu.VMEM((ROWS_PER_W * L,), jnp.int32),
            pltpu.VMEM((ROWS_PER_W * L,), jnp.int32),
            pltpu.VMEM((L,), jnp.int32),
            pltpu.VMEM((BSZ * 4,), jnp.float32),
            pltpu.SMEM((NBLK,), jnp.float32),
            pltpu.SMEM((1,), jnp.int32),
        ],
    )
    return fn(step_v, lprobs, scores_flat)


def kernel(step, lprobs, scores):
    step_v = jnp.broadcast_to(
        jnp.asarray(step, jnp.int32).reshape(()), (L,))
    lp0 = lprobs[:, 0, :].reshape(BSZ * VOCAB)
    sc, ix, bm = _sc_topk(step_v, lp0, scores.reshape(-1))
    return (sc.reshape(BSZ, L)[:, :VK], ix.reshape(BSZ, L)[:, :VK],
            bm.reshape(BSZ, L)[:, :VK])


# speedup vs baseline: 2.4763x; 1.4350x over previous
import jax
import jax.numpy as jnp
from jax import lax
from jax.experimental import pallas as pl
from jax.experimental.pallas import tpu as pltpu
from jax.experimental.pallas import tpu_sc as plsc

NC = 2
NS = 16
NW = NC * NS
L = 16

BSZ = 128
VOCAB = 100000
VK = 8
ROWS_PER_W = BSZ // NW
NCHUNK = VOCAB // L
CPB = 50
NBLK = NCHUNK // CPB
BLK = CPB * L
MAX_SLOTS = 1024

NEG_INF = float("-inf")
BIG_I32 = 2**31 - 1
GROUPS = [(0, 16), (16, 16), (32, 16), (48, 2)]


def _topk_body(step_hbm, lprobs_hbm, scores_hbm,
               out_val_hbm, out_idx_hbm, out_beam_hbm,
               row_v, cidx_v, st_val, st_idx, st_beam,
               step_v, scores_v, bm_smem, slot_smem):
    wid = lax.axis_index("s") * NC + lax.axis_index("c")
    lane = lax.iota(jnp.int32, L)
    minus_inf = jnp.full((L,), NEG_INF, jnp.float32)
    plus_inf = jnp.full((L,), float("inf"), jnp.float32)
    big_vec = jnp.full((L,), BIG_I32, jnp.int32)
    zero_i = jnp.zeros((L,), jnp.int32)

    pltpu.sync_copy(step_hbm, step_v)
    pltpu.sync_copy(scores_hbm, scores_v)
    ssum = jnp.zeros((L,), jnp.float32)
    for i in range(BSZ * 4 // L):
        ssum = ssum + scores_v[pl.ds(i * L, L)]
    mean = jnp.sum(ssum) * (1.0 / (BSZ * 4))
    stepf = jnp.max(step_v[...].astype(jnp.float32))
    shift = stepf * mean

    def row_body(rr, row_carry):
        r = wid * ROWS_PER_W + rr
        with jax.named_scope("row_dma"):
            pltpu.sync_copy(lprobs_hbm.at[r], row_v)
        v0 = row_v[pl.ds(0, L)]
        row_v[pl.ds(0, L)] = jnp.where(lane == 0, minus_inf, v0)

        with jax.named_scope("phase1"):
            def blk_body(b, rowacc):
                base = b * BLK
                acc = minus_inf
                for j in range(CPB):
                    acc = jnp.maximum(acc, row_v[pl.ds(base + j * L, L)])
                bm_smem[b] = jnp.max(acc)
                return jnp.maximum(rowacc, acc)

            rowacc = lax.fori_loop(0, NBLK, blk_body, minus_inf)

        srt = plsc.sort_key_val(rowacc, lane, descending=True)[0]
        t = jnp.min(jnp.where(lane < VK, srt, plus_inf))
        t_vec = jnp.full((L,), t, jnp.float32)

        slot_smem[0] = 0

        with jax.named_scope("phase2"):
            def p2_body(b, carry):
                @pl.when(bm_smem[b] >= t)
                def _visit():
                    cbase = b * CPB
                    for g0, gsz in GROUPS:
                        hits = zero_i
                        for j in range(gsz):
                            v = row_v[pl.ds((cbase + g0 + j) * L, L)]
                            cnt = plsc.all_reduce_population_count(
                                v >= t_vec)
                            hits = jnp.where(lane == j, cnt, hits)
                        hmask = hits > 0
                        s = slot_smem[0]

                        @pl.when(s < MAX_SLOTS - L)
                        def _st():
                            plsc.store_compressed(
                                cidx_v.at[pl.ds(s, L)],
                                lane + (cbase + g0), mask=hmask)
                            slot_smem[0] = s + jnp.max(
                                plsc.all_reduce_population_count(hmask))
                return carry

            lax.fori_loop(0, NBLK, p2_body, 0)
        nslots = slot_smem[0]
        cidx_v[pl.ds(nslots, L)] = zero_i
        ngroups = (nslots + L - 1) // L

        ph3 = jax.named_scope("phase3")
        ph3.__enter__()
        ovv = minus_inf
        oiv = zero_i
        for k in range(VK):
            def mx_body(g, macc):
                base16 = cidx_v[pl.ds(g * L, L)] * L
                for j in range(L):
                    macc = jnp.maximum(
                        macc, plsc.load_gather(row_v, [base16 + j]))
                return macc

            m = jnp.max(lax.fori_loop(0, ngroups, mx_body, minus_inf))
            m_vec = jnp.full((L,), m, jnp.float32)

            def ix_body(g, iacc):
                base16 = cidx_v[pl.ds(g * L, L)] * L
                for j in range(L):
                    xj = plsc.load_gather(row_v, [base16 + j])
                    iacc = jnp.minimum(
                        iacc, jnp.where(xj == m_vec, base16 + j, big_vec))
                return iacc

            ci = jnp.min(lax.fori_loop(0, ngroups, ix_body, big_vec))
            ci_vec = jnp.full((L,), ci, jnp.int32)

            cch = ci // L
            vb = row_v[pl.ds(cch * L, L)]
            row_v[pl.ds(cch * L, L)] = jnp.where(
                cch * L + lane == ci_vec, minus_inf, vb)

            ovv = jnp.where(lane == k, m_vec, ovv)
            oiv = jnp.where(lane == k, ci_vec, oiv)

        sh_vec = jnp.full((L,), shift, jnp.float32)
        st_val[pl.ds(rr * L, L)] = ovv + sh_vec
        st_idx[pl.ds(rr * L, L)] = oiv
        st_beam[pl.ds(rr * L, L)] = zero_i
        ph3.__exit__(None, None, None)
        return row_carry

    lax.fori_loop(0, ROWS_PER_W, row_body, 0)

    base = wid * ROWS_PER_W * L
    pltpu.sync_copy(st_val, out_val_hbm.at[pl.ds(base, ROWS_PER_W * L)])
    pltpu.sync_copy(st_idx, out_idx_hbm.at[pl.ds(base, ROWS_PER_W * L)])
    pltpu.sync_copy(st_beam, out_beam_hbm.at[pl.ds(base, ROWS_PER_W * L)])


@jax.jit
def _sc_topk(step_v, lprobs, scores_flat):
    mesh = plsc.VectorSubcoreMesh(
        core_axis_name="c", subcore_axis_name="s",
        num_cores=NC, num_subcores=NS)
    fn = pl.kernel(
        _topk_body,
        out_type=(
            jax.ShapeDtypeStruct((BSZ * L,), jnp.float32),
            jax.ShapeDtypeStruct((BSZ * L,), jnp.int32),
            jax.ShapeDtypeStruct((BSZ * L,), jnp.int32),
        ),
        mesh=mesh,
        compiler_params=pltpu.CompilerParams(
            needs_layout_passes=False, use_tc_tiling_on_sc=True),
        scratch_types=[
            pltpu.VMEM((VOCAB,), jnp.float32),
            pltpu.VMEM((MAX_SLOTS + L,), jnp.int32),
            pltpu.VMEM((ROWS_PER_W * L,), jnp.float32),
            pltpu.VMEM((ROWS_PER_W * L,), jnp.int32),
            pltpu.VMEM((ROWS_PER_W * L,), jnp.int32),
            pltpu.VMEM((L,), jnp.int32),
            pltpu.VMEM((BSZ * 4,), jnp.float32),
            pltpu.SMEM((NBLK,), jnp.float32),
            pltpu.SMEM((1,), jnp.int32),
        ],
    )
    return fn(step_v, lprobs, scores_flat)


def kernel(step, lprobs, scores):
    step_v = jnp.broadcast_to(
        jnp.asarray(step, jnp.int32).reshape(()), (L,))
    lp0 = lprobs[:, 0, :]
    sc, ix, bm = _sc_topk(step_v, lp0, scores.reshape(-1))
    return (sc.reshape(BSZ, L)[:, :VK], ix.reshape(BSZ, L)[:, :VK],
            bm.reshape(BSZ, L)[:, :VK])
